# Initial kernel scaffold; baseline (speedup 1.0000x reference)
import jax
import jax.numpy as jnp
from jax.experimental import pallas as pl


def kernel(x, W, gamma, beta):
    raise NotImplementedError("write your pallas kernel here")


# speedup vs baseline: 6.4201x; 6.4201x over previous
import functools

import jax
import jax.numpy as jnp
from jax.experimental import pallas as pl
from jax.experimental.pallas import tpu as pltpu

_K = 20
_NEG_INF = float("-inf")


def _pass1_body(x_full_ref, x_rows_ref, w_ref, maxv_ref, minv_ref, stats_ref,
                *, n_rows, n_points, k):
    b = pl.program_id(0)
    j = pl.program_id(1)

    xb = x_full_ref[0]
    xr = x_rows_ref[0]
    w1 = w_ref[:, :3]
    wd = w_ref[:, 3:] - w1

    xxb = jnp.sum(xb * xb, axis=0, keepdims=True)
    xxr = jnp.sum(xr * xr, axis=0, keepdims=True)

    g = jax.lax.dot_general(xr, xb, (((0,), (0,)), ((), ())),
                            preferred_element_type=jnp.float32)
    d = 2.0 * g - xxr.T - xxb

    z = jax.lax.dot_general(xr, wd, (((0,), (1,)), ((), ())),
                            preferred_element_type=jnp.float32)

    iota = jax.lax.broadcasted_iota(jnp.int32, (n_rows, n_points), 1)

    def body(_, carry):
        d, mx, mn, s1, s2 = carry
        am = jnp.argmax(d, axis=1)
        oh = iota == am[:, None]
        ohf = oh.astype(jnp.float32)
        sel = jax.lax.dot_general(ohf, xb, (((1,), (1,)), ((), ())),
                                  preferred_element_type=jnp.float32)
        out_t = jax.lax.dot_general(sel, w1, (((1,), (1,)), ((), ())),
                                    preferred_element_type=jnp.float32) + z
        mx = jnp.maximum(mx, out_t)
        mn = jnp.minimum(mn, out_t)
        s1 = s1 + jnp.sum(out_t, axis=0, keepdims=True)
        s2 = s2 + jnp.sum(out_t * out_t, axis=0, keepdims=True)
        d = jnp.where(oh, _NEG_INF, d)
        return d, mx, mn, s1, s2

    mx0 = jnp.full((n_rows, 64), _NEG_INF, dtype=jnp.float32)
    mn0 = jnp.full((n_rows, 64), jnp.inf, dtype=jnp.float32)
    s0 = jnp.zeros((1, 64), dtype=jnp.float32)
    _, mx, mn, s1, s2 = jax.lax.fori_loop(0, k, body, (d, mx0, mn0, s0, s0))

    maxv_ref[0] = mx
    minv_ref[0] = mn

    @pl.when((b == 0) & (j == 0))
    def _():
        stats_ref[...] = jnp.zeros_like(stats_ref)

    upd = jnp.concatenate([s1, s2, jnp.zeros((6, 64), jnp.float32)], axis=0)
    stats_ref[...] += upd


def _pass2_body(stats_ref, gamma_ref, beta_ref, maxv_ref, minv_ref, out_ref,
                *, count):
    s1 = stats_ref[0:1, :]
    s2 = stats_ref[1:2, :]
    mean = s1 / count
    var = s2 / count - mean * mean
    a = gamma_ref[...] * jax.lax.rsqrt(var + 1e-5)
    c = beta_ref[...] - mean * a
    m = jnp.where(a >= 0.0, maxv_ref[0], minv_ref[0])
    o = a * m + c
    o = jnp.where(o > 0.0, o, 0.2 * o)
    out_ref[0] = o.T


@jax.jit
def kernel(x, W, gamma, beta):
    B, C, N = x.shape
    O = W.shape[0]
    R = 256
    nb = N // R

    grid = (B, nb)
    maxv, minv, stats = pl.pallas_call(
        functools.partial(_pass1_body, n_rows=R, n_points=N, k=_K),
        grid=grid,
        in_specs=[
            pl.BlockSpec((1, C, N), lambda b, j: (b, 0, 0)),
            pl.BlockSpec((1, C, R), lambda b, j: (b, 0, j)),
            pl.BlockSpec((O, 2 * C), lambda b, j: (0, 0)),
        ],
        out_specs=[
            pl.BlockSpec((1, R, O), lambda b, j: (b, j, 0)),
            pl.BlockSpec((1, R, O), lambda b, j: (b, j, 0)),
            pl.BlockSpec((8, O), lambda b, j: (0, 0)),
        ],
        out_shape=[
            jax.ShapeDtypeStruct((B, N, O), jnp.float32),
            jax.ShapeDtypeStruct((B, N, O), jnp.float32),
            jax.ShapeDtypeStruct((8, O), jnp.float32),
        ],
        compiler_params=pltpu.CompilerParams(
            dimension_semantics=("arbitrary", "arbitrary")),
    )(x, x, W)

    count = float(B * N * _K)
    out = pl.pallas_call(
        functools.partial(_pass2_body, count=count),
        grid=grid,
        in_specs=[
            pl.BlockSpec((8, O), lambda b, j: (0, 0)),
            pl.BlockSpec((1, O), lambda b, j: (0, 0)),
            pl.BlockSpec((1, O), lambda b, j: (0, 0)),
            pl.BlockSpec((1, R, O), lambda b, j: (b, j, 0)),
            pl.BlockSpec((1, R, O), lambda b, j: (b, j, 0)),
        ],
        out_specs=pl.BlockSpec((1, O, R), lambda b, j: (b, 0, j)),
        out_shape=jax.ShapeDtypeStruct((B, O, N), jnp.float32),
    )(stats, gamma.reshape(1, O), beta.reshape(1, O), maxv, minv)
    return out


# speedup vs baseline: 8.0755x; 1.2578x over previous
import functools

import jax
import jax.numpy as jnp
from jax.experimental import pallas as pl
from jax.experimental.pallas import tpu as pltpu

_K = 20
_NEG_INF = float("-inf")


def _pass1_body(x_full_ref, x_rows_ref, w_ref, maxv_ref, minv_ref, stats_ref,
                *, n_rows, n_points, k):
    b = pl.program_id(0)
    j = pl.program_id(1)

    xb = x_full_ref[0]
    xr = x_rows_ref[0]
    w1 = w_ref[:, :3]
    wd = w_ref[:, 3:] - w1

    xxb = jnp.sum(xb * xb, axis=0, keepdims=True)
    xxr = jnp.sum(xr * xr, axis=0, keepdims=True)

    g = jax.lax.dot_general(xr, xb, (((0,), (0,)), ((), ())),
                            preferred_element_type=jnp.float32)
    d = 2.0 * g - xxr.T - xxb

    z = jax.lax.dot_general(xr, wd, (((0,), (1,)), ((), ())),
                            preferred_element_type=jnp.float32)
    yt = jax.lax.dot_general(xb, w1, (((0,), (1,)), ((), ())),
                             preferred_element_type=jnp.float32)

    iota = jax.lax.broadcasted_iota(jnp.int32, (n_rows, n_points), 1)

    def body(_, carry):
        d, mx, mn, s1, s2 = carry
        am = jnp.argmax(d, axis=1)
        ohf = jnp.where(iota == am[:, None], 1.0, 0.0)
        out_t = jax.lax.dot_general(ohf, yt, (((1,), (0,)), ((), ())),
                                    preferred_element_type=jnp.float32) + z
        mx = jnp.maximum(mx, out_t)
        mn = jnp.minimum(mn, out_t)
        s1 = s1 + jnp.sum(out_t, axis=0, keepdims=True)
        s2 = s2 + jnp.sum(out_t * out_t, axis=0, keepdims=True)
        d = d - ohf * jnp.float32(3.4e38)
        return d, mx, mn, s1, s2

    mx0 = jnp.full((n_rows, 64), _NEG_INF, dtype=jnp.float32)
    mn0 = jnp.full((n_rows, 64), jnp.inf, dtype=jnp.float32)
    s0 = jnp.zeros((1, 64), dtype=jnp.float32)
    _, mx, mn, s1, s2 = jax.lax.fori_loop(0, k, body, (d, mx0, mn0, s0, s0),
                                          unroll=2)

    maxv_ref[0] = mx
    minv_ref[0] = mn

    @pl.when((b == 0) & (j == 0))
    def _():
        stats_ref[...] = jnp.zeros_like(stats_ref)

    upd = jnp.concatenate([s1, s2, jnp.zeros((6, 64), jnp.float32)], axis=0)
    stats_ref[...] += upd


def _pass2_body(stats_ref, gamma_ref, beta_ref, maxv_ref, minv_ref, out_ref,
                *, count):
    s1 = stats_ref[0:1, :]
    s2 = stats_ref[1:2, :]
    mean = s1 / count
    var = s2 / count - mean * mean
    a = gamma_ref[...] * jax.lax.rsqrt(var + 1e-5)
    c = beta_ref[...] - mean * a
    m = jnp.where(a >= 0.0, maxv_ref[0], minv_ref[0])
    o = a * m + c
    o = jnp.where(o > 0.0, o, 0.2 * o)
    out_ref[0] = o.T


@jax.jit
def kernel(x, W, gamma, beta):
    B, C, N = x.shape
    O = W.shape[0]
    R = 512
    nb = N // R

    grid = (B, nb)
    maxv, minv, stats = pl.pallas_call(
        functools.partial(_pass1_body, n_rows=R, n_points=N, k=_K),
        grid=grid,
        in_specs=[
            pl.BlockSpec((1, C, N), lambda b, j: (b, 0, 0)),
            pl.BlockSpec((1, C, R), lambda b, j: (b, 0, j)),
            pl.BlockSpec((O, 2 * C), lambda b, j: (0, 0)),
        ],
        out_specs=[
            pl.BlockSpec((1, R, O), lambda b, j: (b, j, 0)),
            pl.BlockSpec((1, R, O), lambda b, j: (b, j, 0)),
            pl.BlockSpec((8, O), lambda b, j: (0, 0)),
        ],
        out_shape=[
            jax.ShapeDtypeStruct((B, N, O), jnp.float32),
            jax.ShapeDtypeStruct((B, N, O), jnp.float32),
            jax.ShapeDtypeStruct((8, O), jnp.float32),
        ],
        compiler_params=pltpu.CompilerParams(
            dimension_semantics=("arbitrary", "arbitrary")),
    )(x, x, W)

    count = float(B * N * _K)
    out = pl.pallas_call(
        functools.partial(_pass2_body, count=count),
        grid=grid,
        in_specs=[
            pl.BlockSpec((8, O), lambda b, j: (0, 0)),
            pl.BlockSpec((1, O), lambda b, j: (0, 0)),
            pl.BlockSpec((1, O), lambda b, j: (0, 0)),
            pl.BlockSpec((1, R, O), lambda b, j: (b, j, 0)),
            pl.BlockSpec((1, R, O), lambda b, j: (b, j, 0)),
        ],
        out_specs=pl.BlockSpec((1, O, R), lambda b, j: (b, 0, j)),
        out_shape=jax.ShapeDtypeStruct((B, O, N), jnp.float32),
    )(stats, gamma.reshape(1, O), beta.reshape(1, O), maxv, minv)
    return out


# speedup vs baseline: 10.3032x; 1.2759x over previous
import functools

import jax
import jax.numpy as jnp
from jax.experimental import pallas as pl
from jax.experimental.pallas import tpu as pltpu

_K = 20
_NEG_INF = float("-inf")


def _pass1_body(x_full_ref, x_rows_ref, w_ref, maxv_ref, minv_ref, stats_ref,
                *, n_rows, n_points, k):
    b = pl.program_id(0)
    j = pl.program_id(1)

    xb = x_full_ref[0]
    xr = x_rows_ref[0]
    w1 = w_ref[:, :3]
    wd = w_ref[:, 3:] - w1

    xxb = jnp.sum(xb * xb, axis=0, keepdims=True)
    xxr = jnp.sum(xr * xr, axis=0, keepdims=True)

    g = jax.lax.dot_general(xr, xb, (((0,), (0,)), ((), ())),
                            preferred_element_type=jnp.float32)
    d = 2.0 * g - xxr.T - xxb

    z = jax.lax.dot_general(xr, wd, (((0,), (1,)), ((), ())),
                            preferred_element_type=jnp.float32)
    yt = jax.lax.dot_general(xb, w1, (((0,), (1,)), ((), ())),
                             preferred_element_type=jnp.float32)

    iota = jax.lax.broadcasted_iota(jnp.int32, (n_rows, n_points), 1)

    self_idx = j * n_rows + jax.lax.broadcasted_iota(jnp.int32, (n_rows, 1), 0)

    def step(am_prev, d, mx, mn, s1, s2):
        ohm = iota == am_prev
        ohf = jnp.where(ohm, 1.0, 0.0)
        d = jnp.where(ohm, _NEG_INF, d)
        out_p = jax.lax.dot_general(ohf, yt, (((1,), (0,)), ((), ())),
                                    preferred_element_type=jnp.float32) + z
        mx = jnp.maximum(mx, out_p)
        mn = jnp.minimum(mn, out_p)
        s1 = s1 + jnp.sum(out_p, axis=0, keepdims=True)
        s2 = s2 + jnp.sum(out_p * out_p, axis=0, keepdims=True)
        am = jnp.argmax(d, axis=1)[:, None]
        return am, d, mx, mn, s1, s2

    def body(_, carry):
        return step(*carry)

    mx0 = jnp.full((n_rows, 64), _NEG_INF, dtype=jnp.float32)
    mn0 = jnp.full((n_rows, 64), jnp.inf, dtype=jnp.float32)
    s0 = jnp.zeros((1, 64), dtype=jnp.float32)
    am, d, mx, mn, s1, s2 = jax.lax.fori_loop(
        0, k - 1, body, (self_idx, d, mx0, mn0, s0, s0), unroll=2)

    ohf = jnp.where(iota == am, 1.0, 0.0)
    out_p = jax.lax.dot_general(ohf, yt, (((1,), (0,)), ((), ())),
                                preferred_element_type=jnp.float32) + z
    mx = jnp.maximum(mx, out_p)
    mn = jnp.minimum(mn, out_p)
    s1 = s1 + jnp.sum(out_p, axis=0, keepdims=True)
    s2 = s2 + jnp.sum(out_p * out_p, axis=0, keepdims=True)

    maxv_ref[0] = mx
    minv_ref[0] = mn

    @pl.when((b == 0) & (j == 0))
    def _():
        stats_ref[...] = jnp.zeros_like(stats_ref)

    upd = jnp.concatenate([s1, s2, jnp.zeros((6, 64), jnp.float32)], axis=0)
    stats_ref[...] += upd


def _pass2_body(stats_ref, gamma_ref, beta_ref, maxv_ref, minv_ref, out_ref,
                *, count):
    s1 = stats_ref[0:1, :]
    s2 = stats_ref[1:2, :]
    mean = s1 / count
    var = s2 / count - mean * mean
    a = gamma_ref[...] * jax.lax.rsqrt(var + 1e-5)
    c = beta_ref[...] - mean * a
    m = jnp.where(a >= 0.0, maxv_ref[0], minv_ref[0])
    o = a * m + c
    o = jnp.where(o > 0.0, o, 0.2 * o)
    out_ref[0] = o.T


@jax.jit
def kernel(x, W, gamma, beta):
    B, C, N = x.shape
    O = W.shape[0]
    R = 512
    nb = N // R

    grid = (B, nb)
    maxv, minv, stats = pl.pallas_call(
        functools.partial(_pass1_body, n_rows=R, n_points=N, k=_K),
        grid=grid,
        in_specs=[
            pl.BlockSpec((1, C, N), lambda b, j: (b, 0, 0)),
            pl.BlockSpec((1, C, R), lambda b, j: (b, 0, j)),
            pl.BlockSpec((O, 2 * C), lambda b, j: (0, 0)),
        ],
        out_specs=[
            pl.BlockSpec((1, R, O), lambda b, j: (b, j, 0)),
            pl.BlockSpec((1, R, O), lambda b, j: (b, j, 0)),
            pl.BlockSpec((8, O), lambda b, j: (0, 0)),
        ],
        out_shape=[
            jax.ShapeDtypeStruct((B, N, O), jnp.float32),
            jax.ShapeDtypeStruct((B, N, O), jnp.float32),
            jax.ShapeDtypeStruct((8, O), jnp.float32),
        ],
        compiler_params=pltpu.CompilerParams(
            dimension_semantics=("arbitrary", "arbitrary")),
    )(x, x, W)

    count = float(B * N * _K)
    out = pl.pallas_call(
        functools.partial(_pass2_body, count=count),
        grid=grid,
        in_specs=[
            pl.BlockSpec((8, O), lambda b, j: (0, 0)),
            pl.BlockSpec((1, O), lambda b, j: (0, 0)),
            pl.BlockSpec((1, O), lambda b, j: (0, 0)),
            pl.BlockSpec((1, R, O), lambda b, j: (b, j, 0)),
            pl.BlockSpec((1, R, O), lambda b, j: (b, j, 0)),
        ],
        out_specs=pl.BlockSpec((1, O, R), lambda b, j: (b, 0, j)),
        out_shape=jax.ShapeDtypeStruct((B, O, N), jnp.float32),
    )(stats, gamma.reshape(1, O), beta.reshape(1, O), maxv, minv)
    return out


# speedup vs baseline: 12.5919x; 1.2221x over previous
import functools

import jax
import jax.numpy as jnp
from jax.experimental import pallas as pl
from jax.experimental.pallas import tpu as pltpu

_K = 20
_NEG_INF = float("-inf")


def _pass1_body(x_full_ref, x_rows_ref, w_ref, maxv_ref, minv_ref, stats_ref,
                *, n_rows, n_points, k):
    b = pl.program_id(0)
    j = pl.program_id(1)

    xb = x_full_ref[0]
    xr = x_rows_ref[0]
    w1 = w_ref[:, :3]
    wd = w_ref[:, 3:] - w1

    xxb = jnp.sum(xb * xb, axis=0, keepdims=True)
    xxr = jnp.sum(xr * xr, axis=0, keepdims=True)

    g = jax.lax.dot_general(xr, xb, (((0,), (0,)), ((), ())),
                            preferred_element_type=jnp.float32)
    d = 2.0 * g - xxr.T - xxb

    z = jax.lax.dot_general(xr, wd, (((0,), (1,)), ((), ())),
                            preferred_element_type=jnp.float32)
    yt = jax.lax.dot_general(xb, w1, (((0,), (1,)), ((), ())),
                             preferred_element_type=jnp.float32)

    iota = jax.lax.broadcasted_iota(jnp.int32, (n_rows, n_points), 1)

    self_idx = j * n_rows + jax.lax.broadcasted_iota(jnp.int32, (n_rows, 1), 0)

    def step(am_prev, d, mx, mn, s1, s2):
        ohm = iota == am_prev
        ohf = jnp.where(ohm, 1.0, 0.0)
        d = jnp.where(ohm, _NEG_INF, d)
        out_p = jax.lax.dot_general(ohf, yt, (((1,), (0,)), ((), ())),
                                    preferred_element_type=jnp.float32) + z
        mx = jnp.maximum(mx, out_p)
        mn = jnp.minimum(mn, out_p)
        s1 = s1 + jnp.sum(out_p, axis=0, keepdims=True)
        s2 = s2 + jnp.sum(out_p * out_p, axis=0, keepdims=True)
        am = jnp.argmax(d, axis=1)[:, None]
        return am, d, mx, mn, s1, s2

    def body(_, carry):
        return step(*carry)

    mx0 = jnp.full((n_rows, 64), _NEG_INF, dtype=jnp.float32)
    mn0 = jnp.full((n_rows, 64), jnp.inf, dtype=jnp.float32)
    s0 = jnp.zeros((1, 64), dtype=jnp.float32)
    am, d, mx, mn, s1, s2 = jax.lax.fori_loop(
        0, k - 1, body, (self_idx, d, mx0, mn0, s0, s0), unroll=4)

    ohf = jnp.where(iota == am, 1.0, 0.0)
    out_p = jax.lax.dot_general(ohf, yt, (((1,), (0,)), ((), ())),
                                preferred_element_type=jnp.float32) + z
    mx = jnp.maximum(mx, out_p)
    mn = jnp.minimum(mn, out_p)
    s1 = s1 + jnp.sum(out_p, axis=0, keepdims=True)
    s2 = s2 + jnp.sum(out_p * out_p, axis=0, keepdims=True)

    maxv_ref[0] = mx
    minv_ref[0] = mn

    @pl.when((b == 0) & (j == 0))
    def _():
        stats_ref[...] = jnp.zeros_like(stats_ref)

    upd = jnp.concatenate([s1, s2, jnp.zeros((6, 64), jnp.float32)], axis=0)
    stats_ref[...] += upd


def _pass2_body(stats_ref, gamma_ref, beta_ref, maxv_ref, minv_ref, out_ref,
                *, count):
    s1 = stats_ref[0:1, :]
    s2 = stats_ref[1:2, :]
    mean = s1 / count
    var = s2 / count - mean * mean
    a = gamma_ref[...] * jax.lax.rsqrt(var + 1e-5)
    c = beta_ref[...] - mean * a
    m = jnp.where(a >= 0.0, maxv_ref[0], minv_ref[0])
    o = a * m + c
    o = jnp.where(o > 0.0, o, 0.2 * o)
    out_ref[0] = o.T


@jax.jit
def kernel(x, W, gamma, beta):
    B, C, N = x.shape
    O = W.shape[0]
    R = 512
    nb = N // R

    grid = (B, nb)
    maxv, minv, stats = pl.pallas_call(
        functools.partial(_pass1_body, n_rows=R, n_points=N, k=_K),
        grid=grid,
        in_specs=[
            pl.BlockSpec((1, C, N), lambda b, j: (b, 0, 0)),
            pl.BlockSpec((1, C, R), lambda b, j: (b, 0, j)),
            pl.BlockSpec((O, 2 * C), lambda b, j: (0, 0)),
        ],
        out_specs=[
            pl.BlockSpec((1, R, O), lambda b, j: (b, j, 0)),
            pl.BlockSpec((1, R, O), lambda b, j: (b, j, 0)),
            pl.BlockSpec((8, O), lambda b, j: (0, 0)),
        ],
        out_shape=[
            jax.ShapeDtypeStruct((B, N, O), jnp.float32),
            jax.ShapeDtypeStruct((B, N, O), jnp.float32),
            jax.ShapeDtypeStruct((8, O), jnp.float32),
        ],
        compiler_params=pltpu.CompilerParams(
            dimension_semantics=("arbitrary", "arbitrary")),
    )(x, x, W)

    count = float(B * N * _K)
    out = pl.pallas_call(
        functools.partial(_pass2_body, count=count),
        grid=grid,
        in_specs=[
            pl.BlockSpec((8, O), lambda b, j: (0, 0)),
            pl.BlockSpec((1, O), lambda b, j: (0, 0)),
            pl.BlockSpec((1, O), lambda b, j: (0, 0)),
            pl.BlockSpec((1, R, O), lambda b, j: (b, j, 0)),
            pl.BlockSpec((1, R, O), lambda b, j: (b, j, 0)),
        ],
        out_specs=pl.BlockSpec((1, O, R), lambda b, j: (b, 0, j)),
        out_shape=jax.ShapeDtypeStruct((B, O, N), jnp.float32),
    )(stats, gamma.reshape(1, O), beta.reshape(1, O), maxv, minv)
    return out


# speedup vs baseline: 14.7001x; 1.1674x over previous
import functools

import jax
import jax.numpy as jnp
from jax.experimental import pallas as pl
from jax.experimental.pallas import tpu as pltpu

_K = 20
_NEG_INF = float("-inf")


def _pass1_body(x_full_ref, x_rows_ref, w_ref, maxv_ref, minv_ref, stats_ref,
                *, n_rows, n_points, k):
    b = pl.program_id(0)
    j = pl.program_id(1)

    xb = x_full_ref[0]
    xr = x_rows_ref[0]
    w1 = w_ref[:, :3]
    wd = w_ref[:, 3:] - w1

    xxb = jnp.sum(xb * xb, axis=0, keepdims=True)
    xxr = jnp.sum(xr * xr, axis=0, keepdims=True)

    g = jax.lax.dot_general(xr, xb, (((0,), (0,)), ((), ())),
                            preferred_element_type=jnp.float32)
    d = 2.0 * g - xxr.T - xxb

    z = jax.lax.dot_general(xr, wd, (((0,), (1,)), ((), ())),
                            preferred_element_type=jnp.float32)
    yt = jax.lax.dot_general(xb, w1, (((0,), (1,)), ((), ())),
                             preferred_element_type=jnp.float32)

    iota = jax.lax.broadcasted_iota(jnp.int32, (n_rows, n_points), 1)

    self_idx = j * n_rows + jax.lax.broadcasted_iota(jnp.int32, (n_rows, 1), 0)

    def step(am_prev, d, mx, mn, s1, s2):
        ohm = iota == am_prev
        ohf = jnp.where(ohm, 1.0, 0.0)
        d = jnp.where(ohm, _NEG_INF, d)
        out_p = jax.lax.dot_general(ohf, yt, (((1,), (0,)), ((), ())),
                                    preferred_element_type=jnp.float32) + z
        mx = jnp.maximum(mx, out_p)
        mn = jnp.minimum(mn, out_p)
        s1 = s1 + jnp.sum(out_p, axis=0, keepdims=True)
        s2 = s2 + jnp.sum(out_p * out_p, axis=0, keepdims=True)
        am = jnp.argmax(d, axis=1)[:, None]
        return am, d, mx, mn, s1, s2

    def body(_, carry):
        return step(*carry)

    mx0 = jnp.full((n_rows, 64), _NEG_INF, dtype=jnp.float32)
    mn0 = jnp.full((n_rows, 64), jnp.inf, dtype=jnp.float32)
    s0 = jnp.zeros((1, 64), dtype=jnp.float32)
    am, d, mx, mn, s1, s2 = jax.lax.fori_loop(
        0, k - 1, body, (self_idx, d, mx0, mn0, s0, s0), unroll=19)

    ohf = jnp.where(iota == am, 1.0, 0.0)
    out_p = jax.lax.dot_general(ohf, yt, (((1,), (0,)), ((), ())),
                                preferred_element_type=jnp.float32) + z
    mx = jnp.maximum(mx, out_p)
    mn = jnp.minimum(mn, out_p)
    s1 = s1 + jnp.sum(out_p, axis=0, keepdims=True)
    s2 = s2 + jnp.sum(out_p * out_p, axis=0, keepdims=True)

    maxv_ref[0] = mx
    minv_ref[0] = mn

    @pl.when((b == 0) & (j == 0))
    def _():
        stats_ref[...] = jnp.zeros_like(stats_ref)

    upd = jnp.concatenate([s1, s2, jnp.zeros((6, 64), jnp.float32)], axis=0)
    stats_ref[...] += upd


def _pass2_body(stats_ref, gamma_ref, beta_ref, maxv_ref, minv_ref, out_ref,
                *, count):
    s1 = stats_ref[0:1, :]
    s2 = stats_ref[1:2, :]
    mean = s1 / count
    var = s2 / count - mean * mean
    a = gamma_ref[...] * jax.lax.rsqrt(var + 1e-5)
    c = beta_ref[...] - mean * a
    m = jnp.where(a >= 0.0, maxv_ref[0], minv_ref[0])
    o = a * m + c
    o = jnp.where(o > 0.0, o, 0.2 * o)
    out_ref[0] = o.T


@jax.jit
def kernel(x, W, gamma, beta):
    B, C, N = x.shape
    O = W.shape[0]
    R = 512
    nb = N // R

    grid = (B, nb)
    maxv, minv, stats = pl.pallas_call(
        functools.partial(_pass1_body, n_rows=R, n_points=N, k=_K),
        grid=grid,
        in_specs=[
            pl.BlockSpec((1, C, N), lambda b, j: (b, 0, 0)),
            pl.BlockSpec((1, C, R), lambda b, j: (b, 0, j)),
            pl.BlockSpec((O, 2 * C), lambda b, j: (0, 0)),
        ],
        out_specs=[
            pl.BlockSpec((1, R, O), lambda b, j: (b, j, 0)),
            pl.BlockSpec((1, R, O), lambda b, j: (b, j, 0)),
            pl.BlockSpec((8, O), lambda b, j: (0, 0)),
        ],
        out_shape=[
            jax.ShapeDtypeStruct((B, N, O), jnp.float32),
            jax.ShapeDtypeStruct((B, N, O), jnp.float32),
            jax.ShapeDtypeStruct((8, O), jnp.float32),
        ],
        compiler_params=pltpu.CompilerParams(
            dimension_semantics=("arbitrary", "arbitrary")),
    )(x, x, W)

    count = float(B * N * _K)
    out = pl.pallas_call(
        functools.partial(_pass2_body, count=count),
        grid=grid,
        in_specs=[
            pl.BlockSpec((8, O), lambda b, j: (0, 0)),
            pl.BlockSpec((1, O), lambda b, j: (0, 0)),
            pl.BlockSpec((1, O), lambda b, j: (0, 0)),
            pl.BlockSpec((1, R, O), lambda b, j: (b, j, 0)),
            pl.BlockSpec((1, R, O), lambda b, j: (b, j, 0)),
        ],
        out_specs=pl.BlockSpec((1, O, R), lambda b, j: (b, 0, j)),
        out_shape=jax.ShapeDtypeStruct((B, O, N), jnp.float32),
    )(stats, gamma.reshape(1, O), beta.reshape(1, O), maxv, minv)
    return out
